# baseline (speedup 1.0000x reference)
import functools

import jax
import jax.numpy as jnp
from jax.experimental import pallas as pl

N = 10000
E = 160000
E_LG = 640000
B = 256
D = 128
N_ITER = 4


def _mm_body(a_ref, b_ref, o_ref):
    o_ref[...] = jnp.dot(a_ref[...], b_ref[...],
                         preferred_element_type=jnp.float32)


def _mm(a, b, blk_rows):
    m, k = a.shape
    _, n = b.shape
    assert m % blk_rows == 0
    return pl.pallas_call(
        _mm_body,
        grid=(m // blk_rows,),
        in_specs=[pl.BlockSpec((blk_rows, k), lambda i: (i, 0)),
                  pl.BlockSpec((k, n), lambda i: (0, 0))],
        out_specs=pl.BlockSpec((blk_rows, n), lambda i: (i, 0)),
        out_shape=jax.ShapeDtypeStruct((m, n), jnp.float32),
    )(a, b)


def _mlp_body(x_ref, w1_ref, b1_ref, w2_ref, b2_ref, o_ref):
    h = jnp.maximum(
        jnp.dot(x_ref[...], w1_ref[...], preferred_element_type=jnp.float32)
        + b1_ref[...], 0.0)
    o_ref[...] = jnp.dot(h, w2_ref[...],
                         preferred_element_type=jnp.float32) + b2_ref[...]


def _mlp(x, w1, b1, w2, b2, blk_rows):
    m, k = x.shape
    _, n2 = w1.shape
    _, n = w2.shape
    return pl.pallas_call(
        _mlp_body,
        grid=(m // blk_rows,),
        in_specs=[pl.BlockSpec((blk_rows, k), lambda i: (i, 0)),
                  pl.BlockSpec((k, n2), lambda i: (0, 0)),
                  pl.BlockSpec((1, n2), lambda i: (0, 0)),
                  pl.BlockSpec((n2, n), lambda i: (0, 0)),
                  pl.BlockSpec((1, n), lambda i: (0, 0))],
        out_specs=pl.BlockSpec((blk_rows, n), lambda i: (i, 0)),
        out_shape=jax.ShapeDtypeStruct((m, n), jnp.float32),
    )(x, w1.reshape(k, n2), b1.reshape(1, n2), w2, b2.reshape(1, n))


def kernel(x, edge_index, edge_attr, line_graph_edge_index, edge_index_batch,
           W_u, W_v, W_edge, W_rel, W_root, b_conv, a, a_bias,
           W_gout, b_gout, W_lb1, b_lb1, W_lb2, b_lb2):
    src, dst = edge_index[0], edge_index[1]
    lg_src, lg_dst = line_graph_edge_index[0], line_graph_edge_index[1]

    gu = _mm(x, W_u, 2000)
    gv = _mm(x, W_v, 2000)
    euv = _mm(edge_attr, W_edge, 8000)
    ea = (gu[src] + gv[dst] + euv) / 3.0

    hs = []
    ms = []
    h = ea
    for t in range(N_ITER + 1):
        m = jax.ops.segment_sum(h[lg_src], lg_dst, num_segments=E)
        ms.append(m)
        if t < N_ITER:
            h = ea + m
            hs.append(h)

    gembs = []
    for t in range(N_ITER):
        x_conv = ms[t + 1] @ W_rel + hs[t] @ W_root + b_conv
        mx = jax.ops.segment_max(x_conv, edge_index_batch, num_segments=B)
        mx = jnp.where(jnp.isfinite(mx), mx, 0.0)
        ex = jnp.exp(x_conv - mx[edge_index_batch])
        denom = jax.ops.segment_sum(ex, edge_index_batch, num_segments=B)
        scores = ex / (denom[edge_index_batch] + 1e-16)
        gembs.append(jax.ops.segment_sum(hs[t] * scores, edge_index_batch,
                                         num_segments=B))

    gemb_all = jnp.concatenate(gembs, axis=0)
    gout_all = jnp.tanh(_mm(gemb_all, W_gout, B) + b_gout)
    gout_all = gout_all.reshape(N_ITER, B, D).transpose(1, 2, 0)
    sc = jnp.sum(gout_all * a, axis=1, keepdims=True) + a_bias
    sc = jax.nn.softmax(sc, axis=-1)
    sc_e = sc[edge_index_batch.astype(jnp.int32)][:, 0, :]

    out_sum = jnp.zeros((E, D), jnp.float32)
    for t in range(N_ITER):
        out_sum = out_sum + hs[t] * sc_e[:, t:t + 1]

    xn = x + jax.ops.segment_sum(out_sum, dst, num_segments=N)
    return _mlp(xn, W_lb1, b_lb1, W_lb2, b_lb2, 2000)


# speedup vs baseline: 1.5319x; 1.5319x over previous
import functools

import jax
import jax.numpy as jnp
from jax import lax
from jax.experimental import pallas as pl
from jax.experimental.pallas import tpu as pltpu
from jax.experimental.pallas import tpu_sc as plsc

N = 10000
E = 160000
E_LG = 640000
B = 256
D = 128
N_ITER = 4


K = 128
CH = 10240
NCH = 16
ACC = CH + 256
NB = E_LG // K + NCH


def _seg_body(h_hbm, srcp_hbm, dstp_hbm, base_hbm, z_hbm, m_hbm,
              base_v, idx_v, dloc_v, rows_v, zbuf, acc, sem, sem2):
    cid = lax.axis_index("c")
    sid = lax.axis_index("s")
    pltpu.async_copy(base_hbm, base_v, sem).wait()
    pltpu.async_copy(z_hbm, zbuf, sem).wait()

    def _zero_my_slice():
        r0 = sid * 656
        for o, n in ((0, 128), (128, 128), (256, 128), (384, 128),
                     (512, 128), (640, 16)):
            pltpu.async_copy(zbuf.at[pl.ds(0, n)], acc.at[pl.ds(r0 + o, n)],
                             sem).wait()

    _zero_my_slice()
    plsc.subcore_barrier()

    bv0 = base_v[pl.ds(0, 16)]
    bv1 = base_v[pl.ds(16, 16)]

    def _at(j):
        return bv0[j] if j < 16 else bv1[j - 16]

    for i in range(NCH // 2):
        chunk = 2 * i + cid
        lo = jnp.where(cid == 0, _at(2 * i), _at(2 * i + 1))
        hi = jnp.where(cid == 0, _at(2 * i + 1), _at(2 * i + 2))

        @pl.loop(lo + sid, hi, step=16)
        def _blk(b):
            ci = pltpu.async_copy(srcp_hbm.at[pl.ds(b * K, K)], idx_v, sem)
            cd = pltpu.async_copy(dstp_hbm.at[pl.ds(b * K, K)], dloc_v, sem2)
            ci.wait()
            cd.wait()
            pltpu.async_copy(h_hbm.at[idx_v], rows_v, sem).wait()
            pltpu.async_copy(rows_v, acc.at[dloc_v], sem2, add=True).wait()

        plsc.subcore_barrier()
        out0 = chunk * CH + sid * 640
        a0 = sid * 640
        for o in range(0, 640, 128):
            @pl.when(out0 + o + 128 <= E)
            def _w(o=o):
                pltpu.async_copy(acc.at[pl.ds(a0 + o, 128)], rows_v, sem).wait()
                pltpu.async_copy(rows_v, m_hbm.at[pl.ds(out0 + o, 128)],
                                 sem).wait()
        _zero_my_slice()
        plsc.subcore_barrier()


@jax.jit
def _segsum_sc(h, src_pad, dst_pad, base_pad, zrows):
    fn = pl.kernel(
        _seg_body,
        mesh=plsc.VectorSubcoreMesh(core_axis_name="c", subcore_axis_name="s"),
        out_type=jax.ShapeDtypeStruct((E, D), jnp.float32),
        scratch_types=[
            pltpu.VMEM((32,), jnp.int32),
            pltpu.VMEM((K,), jnp.int32),
            pltpu.VMEM((K,), jnp.int32),
            pltpu.VMEM((K, D), jnp.float32),
            pltpu.VMEM((K, D), jnp.float32),
            pltpu.VMEM_SHARED((ACC, D), jnp.float32),
            pltpu.SemaphoreType.DMA,
            pltpu.SemaphoreType.DMA,
        ],
    )
    return fn(h, src_pad, dst_pad, base_pad, zrows)


def _lg_preprocess(lg_src, lg_dst):
    order = jnp.argsort(lg_dst)
    srcs = lg_src[order]
    dsts = lg_dst[order]
    chunk_starts = jnp.searchsorted(
        dsts, jnp.minimum(jnp.arange(NCH + 1) * CH, E)).astype(jnp.int32)
    n_c = chunk_starts[1:] - chunk_starts[:-1]
    nb_c = (n_c + K - 1) // K
    blk_base = jnp.concatenate([jnp.zeros((1,), jnp.int32), jnp.cumsum(nb_c)]).astype(jnp.int32)
    base_pad = jnp.zeros((32,), jnp.int32).at[:NCH + 1].set(blk_base)
    b = jnp.arange(NB, dtype=jnp.int32)
    c_of_b = jnp.clip(jnp.searchsorted(blk_base, b, side="right") - 1, 0, NCH - 1).astype(jnp.int32)
    slot = (chunk_starts[c_of_b][:, None]
            + (b[:, None] - blk_base[c_of_b][:, None]) * K
            + jnp.arange(K, dtype=jnp.int32)[None, :])
    valid = slot < chunk_starts[c_of_b + 1][:, None]
    src_pad = jnp.where(valid, jnp.take(srcs, slot, mode="clip"), 0).astype(jnp.int32)
    dst_pad = jnp.where(valid, jnp.take(dsts, slot, mode="clip") - c_of_b[:, None] * CH,
                        CH).astype(jnp.int32)
    zrows = jnp.zeros((K, D), jnp.float32)
    return src_pad.reshape(-1), dst_pad.reshape(-1), base_pad, zrows


def _mm_body(a_ref, b_ref, o_ref):
    o_ref[...] = jnp.dot(a_ref[...], b_ref[...],
                         preferred_element_type=jnp.float32)


def _mm(a, b, blk_rows):
    m, k = a.shape
    _, n = b.shape
    assert m % blk_rows == 0
    return pl.pallas_call(
        _mm_body,
        grid=(m // blk_rows,),
        in_specs=[pl.BlockSpec((blk_rows, k), lambda i: (i, 0)),
                  pl.BlockSpec((k, n), lambda i: (0, 0))],
        out_specs=pl.BlockSpec((blk_rows, n), lambda i: (i, 0)),
        out_shape=jax.ShapeDtypeStruct((m, n), jnp.float32),
    )(a, b)


def _mlp_body(x_ref, w1_ref, b1_ref, w2_ref, b2_ref, o_ref):
    h = jnp.maximum(
        jnp.dot(x_ref[...], w1_ref[...], preferred_element_type=jnp.float32)
        + b1_ref[...], 0.0)
    o_ref[...] = jnp.dot(h, w2_ref[...],
                         preferred_element_type=jnp.float32) + b2_ref[...]


def _mlp(x, w1, b1, w2, b2, blk_rows):
    m, k = x.shape
    _, n2 = w1.shape
    _, n = w2.shape
    return pl.pallas_call(
        _mlp_body,
        grid=(m // blk_rows,),
        in_specs=[pl.BlockSpec((blk_rows, k), lambda i: (i, 0)),
                  pl.BlockSpec((k, n2), lambda i: (0, 0)),
                  pl.BlockSpec((1, n2), lambda i: (0, 0)),
                  pl.BlockSpec((n2, n), lambda i: (0, 0)),
                  pl.BlockSpec((1, n), lambda i: (0, 0))],
        out_specs=pl.BlockSpec((blk_rows, n), lambda i: (i, 0)),
        out_shape=jax.ShapeDtypeStruct((m, n), jnp.float32),
    )(x, w1.reshape(k, n2), b1.reshape(1, n2), w2, b2.reshape(1, n))


def kernel(x, edge_index, edge_attr, line_graph_edge_index, edge_index_batch,
           W_u, W_v, W_edge, W_rel, W_root, b_conv, a, a_bias,
           W_gout, b_gout, W_lb1, b_lb1, W_lb2, b_lb2):
    src, dst = edge_index[0], edge_index[1]
    lg_src, lg_dst = line_graph_edge_index[0], line_graph_edge_index[1]

    gu = _mm(x, W_u, 2000)
    gv = _mm(x, W_v, 2000)
    euv = _mm(edge_attr, W_edge, 8000)
    ea = (gu[src] + gv[dst] + euv) / 3.0

    src_pad, dst_pad, base_pad, zrows = _lg_preprocess(lg_src, lg_dst)
    hs = []
    ms = []
    h = ea
    for t in range(N_ITER + 1):
        m = _segsum_sc(h, src_pad, dst_pad, base_pad, zrows)
        ms.append(m)
        if t < N_ITER:
            h = ea + m
            hs.append(h)

    gembs = []
    for t in range(N_ITER):
        x_conv = ms[t + 1] @ W_rel + hs[t] @ W_root + b_conv
        mx = jax.ops.segment_max(x_conv, edge_index_batch, num_segments=B)
        mx = jnp.where(jnp.isfinite(mx), mx, 0.0)
        ex = jnp.exp(x_conv - mx[edge_index_batch])
        denom = jax.ops.segment_sum(ex, edge_index_batch, num_segments=B)
        scores = ex / (denom[edge_index_batch] + 1e-16)
        gembs.append(jax.ops.segment_sum(hs[t] * scores, edge_index_batch,
                                         num_segments=B))

    gemb_all = jnp.concatenate(gembs, axis=0)
    gout_all = jnp.tanh(_mm(gemb_all, W_gout, B) + b_gout)
    gout_all = gout_all.reshape(N_ITER, B, D).transpose(1, 2, 0)
    sc = jnp.sum(gout_all * a, axis=1, keepdims=True) + a_bias
    sc = jax.nn.softmax(sc, axis=-1)
    sc_e = sc[edge_index_batch.astype(jnp.int32)][:, 0, :]

    out_sum = jnp.zeros((E, D), jnp.float32)
    for t in range(N_ITER):
        out_sum = out_sum + hs[t] * sc_e[:, t:t + 1]

    xn = x + jax.ops.segment_sum(out_sum, dst, num_segments=N)
    return _mlp(xn, W_lb1, b_lb1, W_lb2, b_lb2, 2000)
